# baseline (speedup 1.0000x reference)
import functools
import math

import jax
import jax.numpy as jnp
from jax import lax
from jax.experimental import pallas as pl
from jax.experimental.pallas import tpu as pltpu
from jax.experimental.pallas import tpu_sc as plsc

VOCAB = 1_000_000
VPACK = 500_032
D = 64
ROWS = 4096
COLS = 200
NC, NS = 2, 16
NW = NC * NS
BCOL = ROWS // NW
SCALE = math.sqrt(D)

_mesh = plsc.VectorSubcoreMesh(core_axis_name="c", subcore_axis_name="s")


def _pack_body(in_ref, out_ref):
    for i in range(256):
        t = in_ref[:, i * 128:(i + 1) * 128].T
        out_ref[i * 64:(i + 1) * 64, :] = jnp.concatenate(
            [t[0:64, :], t[64:128, :]], axis=1)


def _pack_table(tab_t):
    return pl.pallas_call(
        _pack_body,
        grid=(31,),
        in_specs=[pl.BlockSpec((64, 32768), lambda c: (0, c))],
        out_specs=pl.BlockSpec((16384, 128), lambda c: (c, 0)),
        out_shape=jax.ShapeDtypeStruct((VPACK, 2 * D), jnp.float32),
    )(tab_t)


def _make_gather(cols):
    @functools.partial(
        pl.kernel,
        mesh=_mesh,
        compiler_params=pltpu.CompilerParams(use_tc_tiling_on_sc=False),
        out_type=jax.ShapeDtypeStruct((ROWS * cols, D), jnp.float32),
        scratch_types=[
            pltpu.VMEM((cols, BCOL), jnp.int32),
            pltpu.VMEM((BCOL, D), jnp.float32),
            pltpu.VMEM((BCOL, D), jnp.float32),
            pltpu.VMEM((BCOL, D), jnp.float32),
            pltpu.SemaphoreType.DMA,
            pltpu.SemaphoreType.DMA,
            pltpu.SemaphoreType.DMA,
        ],
    )
    def _gather(xs_hbm, tab_hbm, out_hbm, idx_v, rows0_v, rows1_v, rows2_v,
                sem0, sem1, sem2):
        wid = lax.axis_index("s") * NC + lax.axis_index("c")
        pltpu.sync_copy(xs_hbm.at[:, pl.ds(wid * BCOL, BCOL)], idx_v)

        bufs = (rows0_v, rows1_v, rows2_v)
        sems = (sem0, sem1, sem2)

        def gat(s, b):
            return pltpu.make_async_copy(
                tab_hbm.at[idx_v.at[s]], bufs[b], sems[b])

        def put(s, b):
            base = s * ROWS + wid * BCOL
            pltpu.sync_copy(bufs[b], out_hbm.at[pl.ds(base, BCOL)])

        for b in range(3):
            gat(b, b).start()

        def chunk_body(s3, carry):
            for b in range(3):
                s = s3 * 3 + b
                gat(s, b).wait()
                put(s, b)
                gat(s + 3, b).start()
            return carry

        lax.fori_loop(0, cols // 3 - 1, chunk_body, 0)
        for s in range(3 * (cols // 3 - 1), cols):
            b = s % 3
            gat(s, b).wait()
            put(s, b)
            if s + 3 < cols:
                gat(s + 3, b).start()

    return _gather


def _finish_body(in_ref, out_ref):
    for s in range(4):
        for k in range(NW):
            r = s * NW + k
            x = in_ref[r * D:(r + 1) * D, :]
            t = x.T
            out_ref[s, :, k * BCOL:(k + 1) * BCOL] = jnp.concatenate(
                [t[0:D, :], t[D:2 * D, :]], axis=1) * jnp.float32(SCALE)


def _finish(gathered, cols):
    return pl.pallas_call(
        _finish_body,
        grid=(cols // 4,),
        in_specs=[pl.BlockSpec((ROWS * D * 4 // 128, 128), lambda s: (s, 0))],
        out_specs=pl.BlockSpec((4, D, ROWS), lambda s: (s, 0, 0)),
        out_shape=jax.ShapeDtypeStruct((cols, D, ROWS), jnp.float32),
    )(gathered)


def kernel(x, table):
    xt = x.T.astype(jnp.int32)
    xs = ((xt & ~jnp.int32(127)) | ((xt & 63) << 1) | ((xt >> 6) & 1))
    xs = xs.reshape(COLS, NW, 2, D).transpose(0, 1, 3, 2).reshape(COLS, ROWS)

    tab_t = table.T
    packed = _pack_table(tab_t)
    tab_lin = packed.reshape(-1).reshape(2 * VPACK, D)

    half = COLS // 2
    ys = []
    gfn = _make_gather(half)
    for h in range(2):
        g = gfn(xs[h * half:(h + 1) * half], tab_lin)
        g2 = g.reshape(-1).reshape(ROWS * half * D // 128, 128)
        ys.append(_finish(g2, half))
    y = jnp.concatenate(ys, axis=0)
    return jnp.transpose(y, (2, 0, 1))


# speedup vs baseline: 1.2459x; 1.2459x over previous
import functools
import math

import jax
import jax.numpy as jnp
from jax import lax
from jax.experimental import pallas as pl
from jax.experimental.pallas import tpu as pltpu
from jax.experimental.pallas import tpu_sc as plsc

VOCAB = 1_000_000
VPACK = 500_032
D = 64
ROWS = 4096
COLS = 200
NC, NS = 2, 16
NW = NC * NS
BCOL = ROWS // NW
SCALE = math.sqrt(D)

_mesh = plsc.VectorSubcoreMesh(core_axis_name="c", subcore_axis_name="s")


def _pack_body(in_ref, out_ref):
    for i in range(256):
        t = in_ref[:, i * 128:(i + 1) * 128].T
        out_ref[i * 64:(i + 1) * 64, :] = jnp.concatenate(
            [t[0:64, :], t[64:128, :]], axis=1)


def _pack_table(tab_t):
    return pl.pallas_call(
        _pack_body,
        grid=(31,),
        in_specs=[pl.BlockSpec((64, 32768), lambda c: (0, c))],
        out_specs=pl.BlockSpec((16384, 128), lambda c: (c, 0)),
        out_shape=jax.ShapeDtypeStruct((VPACK, 2 * D), jnp.float32),
    )(tab_t)


def _make_gather(cols):
    @functools.partial(
        pl.kernel,
        mesh=_mesh,
        compiler_params=pltpu.CompilerParams(use_tc_tiling_on_sc=False),
        out_type=jax.ShapeDtypeStruct((ROWS * cols, D), jnp.float32),
        scratch_types=[
            pltpu.VMEM((cols, BCOL), jnp.int32),
            pltpu.VMEM((BCOL, D), jnp.float32),
            pltpu.VMEM((BCOL, D), jnp.float32),
            pltpu.VMEM((BCOL, D), jnp.float32),
            pltpu.SemaphoreType.DMA,
            pltpu.SemaphoreType.DMA,
            pltpu.SemaphoreType.DMA,
        ],
    )
    def _gather(xs_hbm, tab_hbm, out_hbm, idx_v, rows0_v, rows1_v, rows2_v,
                sem0, sem1, sem2):
        wid = lax.axis_index("s") * NC + lax.axis_index("c")
        pltpu.sync_copy(xs_hbm.at[:, pl.ds(wid * BCOL, BCOL)], idx_v)

        bufs = (rows0_v, rows1_v, rows2_v)
        sems = (sem0, sem1, sem2)

        def gat(s, b):
            return pltpu.make_async_copy(
                tab_hbm.at[idx_v.at[s]], bufs[b], sems[b])

        def put(s, b):
            base = s * ROWS + wid * BCOL
            pltpu.sync_copy(bufs[b], out_hbm.at[pl.ds(base, BCOL)])

        for b in range(3):
            gat(b, b).start()

        def chunk_body(s3, carry):
            for b in range(3):
                s = s3 * 3 + b
                gat(s, b).wait()
                put(s, b)
                gat(s + 3, b).start()
            return carry

        lax.fori_loop(0, cols // 3 - 1, chunk_body, 0)
        for s in range(3 * (cols // 3 - 1), cols):
            b = s % 3
            gat(s, b).wait()
            put(s, b)
            if s + 3 < cols:
                gat(s + 3, b).start()

    return _gather


def _finish_body(buf_ref, in_ref, out_ref):
    del buf_ref
    for s in range(4):
        for k in range(NW):
            r = s * NW + k
            x = in_ref[r * D:(r + 1) * D, :]
            t = x.T
            out_ref[s, :, k * BCOL:(k + 1) * BCOL] = jnp.concatenate(
                [t[0:D, :], t[D:2 * D, :]], axis=1) * jnp.float32(SCALE)


def _finish_half(buf, gathered, h, half):
    return pl.pallas_call(
        _finish_body,
        grid=(half // 4,),
        in_specs=[
            pl.BlockSpec(memory_space=pl.ANY),
            pl.BlockSpec((ROWS * D * 4 // 128, 128), lambda s: (s, 0)),
        ],
        out_specs=pl.BlockSpec((4, D, ROWS),
                               lambda s, h=h, half=half: (h * half // 4 + s, 0, 0)),
        out_shape=jax.ShapeDtypeStruct((COLS, D, ROWS), jnp.float32),
        input_output_aliases={0: 0},
    )(buf, gathered)


def _alloc_body(out_ref):
    out_ref[...] = jnp.zeros((4, D, ROWS), jnp.float32)


def _alloc_out():
    return pl.pallas_call(
        _alloc_body,
        grid=(1,),
        out_specs=pl.BlockSpec((4, D, ROWS), lambda i: (0, 0, 0)),
        out_shape=jax.ShapeDtypeStruct((COLS, D, ROWS), jnp.float32),
    )()


def kernel(x, table):
    xt = x.T.astype(jnp.int32)
    xs = ((xt & ~jnp.int32(127)) | ((xt & 63) << 1) | ((xt >> 6) & 1))
    xs = xs.reshape(COLS, NW, 2, D).transpose(0, 1, 3, 2).reshape(COLS, ROWS)

    tab_t = table.T
    packed = _pack_table(tab_t)
    tab_lin = packed.reshape(-1).reshape(2 * VPACK, D)

    half = COLS // 2
    gfn = _make_gather(half)
    y = _alloc_out()
    for h in range(2):
        g = gfn(xs[h * half:(h + 1) * half], tab_lin)
        g2 = g.reshape(-1).reshape(ROWS * half * D // 128, 128)
        y = _finish_half(y, g2, h, half)
    return jnp.transpose(y, (2, 0, 1))


# speedup vs baseline: 1.2477x; 1.0015x over previous
import functools
import math

import jax
import jax.numpy as jnp
from jax import lax
from jax.experimental import pallas as pl
from jax.experimental.pallas import tpu as pltpu
from jax.experimental.pallas import tpu_sc as plsc

VOCAB = 1_000_000
VPACK = 500_032
D = 64
ROWS = 4096
COLS = 200
NC, NS = 2, 16
NW = NC * NS
BCOL = ROWS // NW
SCALE = math.sqrt(D)

_mesh = plsc.VectorSubcoreMesh(core_axis_name="c", subcore_axis_name="s")


def _pack_body(in_ref, out_ref):
    for i in range(256):
        t = in_ref[:, i * 128:(i + 1) * 128].T
        out_ref[i * 64:(i + 1) * 64, :] = jnp.concatenate(
            [t[0:64, :], t[64:128, :]], axis=1)


def _pack_table(tab_t):
    return pl.pallas_call(
        _pack_body,
        grid=(31,),
        in_specs=[pl.BlockSpec((64, 32768), lambda c: (0, c))],
        out_specs=pl.BlockSpec((16384, 128), lambda c: (c, 0)),
        out_shape=jax.ShapeDtypeStruct((VPACK, 2 * D), jnp.float32),
    )(tab_t)


def _make_gather(cols):
    @functools.partial(
        pl.kernel,
        mesh=_mesh,
        compiler_params=pltpu.CompilerParams(use_tc_tiling_on_sc=False),
        out_type=jax.ShapeDtypeStruct((ROWS * cols, D), jnp.float32),
        scratch_types=[
            pltpu.VMEM((cols, BCOL), jnp.int32),
            pltpu.VMEM((BCOL, D), jnp.float32),
            pltpu.VMEM((BCOL, D), jnp.float32),
            pltpu.VMEM((BCOL, D), jnp.float32),
            pltpu.SemaphoreType.DMA,
            pltpu.SemaphoreType.DMA,
            pltpu.SemaphoreType.DMA,
        ],
    )
    def _gather(xs_hbm, tab_hbm, out_hbm, idx_v, rows0_v, rows1_v, rows2_v,
                sem0, sem1, sem2):
        wid = lax.axis_index("s") * NC + lax.axis_index("c")
        pltpu.sync_copy(xs_hbm.at[:, pl.ds(wid * BCOL, BCOL)], idx_v)

        bufs = (rows0_v, rows1_v, rows2_v)
        sems = (sem0, sem1, sem2)

        def gat(s, b):
            return pltpu.make_async_copy(
                tab_hbm.at[idx_v.at[s]], bufs[b], sems[b])

        def put(s, b):
            base = s * ROWS + wid * BCOL
            pltpu.sync_copy(bufs[b], out_hbm.at[pl.ds(base, BCOL)])

        for b in range(3):
            gat(b, b).start()

        def chunk_body(s3, carry):
            for b in range(3):
                s = s3 * 3 + b
                gat(s, b).wait()
                put(s, b)
                gat(s + 3, b).start()
            return carry

        lax.fori_loop(0, cols // 3 - 1, chunk_body, 0)
        for s in range(3 * (cols // 3 - 1), cols):
            b = s % 3
            gat(s, b).wait()
            put(s, b)
            if s + 3 < cols:
                gat(s + 3, b).start()

    return _gather


def _finish_body(buf_ref, in_ref, out_ref):
    del buf_ref
    for s in range(5):
        for k in range(NW):
            r = s * NW + k
            x = in_ref[r * D:(r + 1) * D, :]
            t = x.T
            out_ref[s, :, k * BCOL:(k + 1) * BCOL] = jnp.concatenate(
                [t[0:D, :], t[D:2 * D, :]], axis=1) * jnp.float32(SCALE)


def _finish_half(buf, gathered, h, half):
    return pl.pallas_call(
        _finish_body,
        grid=(half // 5,),
        in_specs=[
            pl.BlockSpec(memory_space=pl.ANY),
            pl.BlockSpec((ROWS * D * 5 // 128, 128), lambda s: (s, 0)),
        ],
        out_specs=pl.BlockSpec((5, D, ROWS),
                               lambda s, h=h, half=half: (h * half // 5 + s, 0, 0)),
        out_shape=jax.ShapeDtypeStruct((COLS, D, ROWS), jnp.float32),
        input_output_aliases={0: 0},
    )(buf, gathered)


def _alloc_body(out_ref):
    out_ref[...] = jnp.zeros((5, D, ROWS), jnp.float32)


def _alloc_out():
    return pl.pallas_call(
        _alloc_body,
        grid=(1,),
        out_specs=pl.BlockSpec((5, D, ROWS), lambda i: (0, 0, 0)),
        out_shape=jax.ShapeDtypeStruct((COLS, D, ROWS), jnp.float32),
    )()


def kernel(x, table):
    xt = x.T.astype(jnp.int32)
    xs = ((xt & ~jnp.int32(127)) | ((xt & 63) << 1) | ((xt >> 6) & 1))
    xs = xs.reshape(COLS, NW, 2, D).transpose(0, 1, 3, 2).reshape(COLS, ROWS)

    tab_t = table.T
    packed = _pack_table(tab_t)
    tab_lin = packed.reshape(-1).reshape(2 * VPACK, D)

    half = COLS // 4
    gfn = _make_gather(half)
    y = _alloc_out()
    for h in range(4):
        g = gfn(xs[h * half:(h + 1) * half], tab_lin)
        g2 = g.reshape(-1).reshape(ROWS * half * D // 128, 128)
        y = _finish_half(y, g2, h, half)
    return jnp.transpose(y, (2, 0, 1))


# speedup vs baseline: 1.2496x; 1.0016x over previous
import functools
import math

import jax
import jax.numpy as jnp
from jax import lax
from jax.experimental import pallas as pl
from jax.experimental.pallas import tpu as pltpu
from jax.experimental.pallas import tpu_sc as plsc

VOCAB = 1_000_000
VPACK = 500_032
D = 64
ROWS = 4096
COLS = 200
NC, NS = 2, 16
NW = NC * NS
BCOL = ROWS // NW
SCALE = math.sqrt(D)

_mesh = plsc.VectorSubcoreMesh(core_axis_name="c", subcore_axis_name="s")


def _pack_body(in_ref, out_ref):
    for i in range(256):
        t = in_ref[:, i * 128:(i + 1) * 128].T
        out_ref[i * 64:(i + 1) * 64, :] = jnp.concatenate(
            [t[0:64, :], t[64:128, :]], axis=1)


def _pack_table(tab_t):
    return pl.pallas_call(
        _pack_body,
        grid=(31,),
        in_specs=[pl.BlockSpec((64, 32768), lambda c: (0, c))],
        out_specs=pl.BlockSpec((16384, 128), lambda c: (c, 0)),
        out_shape=jax.ShapeDtypeStruct((VPACK, 2 * D), jnp.float32),
    )(tab_t)


def _make_gather(cols):
    @functools.partial(
        pl.kernel,
        mesh=_mesh,
        compiler_params=pltpu.CompilerParams(use_tc_tiling_on_sc=False),
        out_type=jax.ShapeDtypeStruct((ROWS * cols, D), jnp.float32),
        scratch_types=[
            pltpu.VMEM((cols, BCOL), jnp.int32),
            pltpu.VMEM((BCOL, D), jnp.float32),
            pltpu.VMEM((BCOL, D), jnp.float32),
            pltpu.VMEM((BCOL, D), jnp.float32),
            pltpu.SemaphoreType.DMA,
            pltpu.SemaphoreType.DMA,
            pltpu.SemaphoreType.DMA,
        ],
    )
    def _gather(xs_hbm, tab_hbm, out_hbm, idx_v, rows0_v, rows1_v, rows2_v,
                sem0, sem1, sem2):
        wid = lax.axis_index("s") * NC + lax.axis_index("c")
        pltpu.sync_copy(xs_hbm.at[:, pl.ds(wid * BCOL, BCOL)], idx_v)

        bufs = (rows0_v, rows1_v, rows2_v)
        sems = (sem0, sem1, sem2)

        def gat(s, b):
            return pltpu.make_async_copy(
                tab_hbm.at[idx_v.at[s]], bufs[b], sems[b])

        def put(s, b):
            base = s * ROWS + wid * BCOL
            pltpu.sync_copy(bufs[b], out_hbm.at[pl.ds(base, BCOL)])

        for b in range(3):
            gat(b, b).start()

        def chunk_body(s3, carry):
            for b in range(3):
                s = s3 * 3 + b
                gat(s, b).wait()
                put(s, b)
                gat(s + 3, b).start()
            return carry

        lax.fori_loop(0, cols // 3 - 1, chunk_body, 0)
        for s in range(3 * (cols // 3 - 1), cols):
            b = s % 3
            gat(s, b).wait()
            put(s, b)
            if s + 3 < cols:
                gat(s + 3, b).start()

    return _gather


def _finish_body(buf_ref, in_ref, out_ref):
    del buf_ref
    for s in range(5):
        for k in range(NW):
            r = s * NW + k
            x = in_ref[r * D:(r + 1) * D, :]
            t = x.T
            out_ref[s, :, k * BCOL:(k + 1) * BCOL] = jnp.concatenate(
                [t[0:D, :], t[D:2 * D, :]], axis=1) * jnp.float32(SCALE)


def _finish_half(buf, gathered, h, half):
    return pl.pallas_call(
        _finish_body,
        grid=(half // 5,),
        in_specs=[
            pl.BlockSpec(memory_space=pl.ANY),
            pl.BlockSpec((ROWS * D * 5 // 128, 128), lambda s: (s, 0)),
        ],
        out_specs=pl.BlockSpec((5, D, ROWS),
                               lambda s, h=h, half=half: (h * half // 5 + s, 0, 0)),
        out_shape=jax.ShapeDtypeStruct((COLS, D, ROWS), jnp.float32),
        input_output_aliases={0: 0},
    )(buf, gathered)


def _alloc_body(out_ref):
    out_ref[...] = jnp.zeros((5, D, ROWS), jnp.float32)


def _alloc_out():
    return pl.pallas_call(
        _alloc_body,
        grid=(1,),
        out_specs=pl.BlockSpec((5, D, ROWS), lambda i: (0, 0, 0)),
        out_shape=jax.ShapeDtypeStruct((COLS, D, ROWS), jnp.float32),
    )()


def kernel(x, table):
    xt = x.T.astype(jnp.int32)
    xs = ((xt & ~jnp.int32(127)) | ((xt & 63) << 1) | ((xt >> 6) & 1))
    xs = xs.reshape(COLS, NW, 2, D).transpose(0, 1, 3, 2).reshape(COLS, ROWS)

    tab_t = table.T
    packed = _pack_table(tab_t)
    tab_lin = packed.reshape(-1).reshape(2 * VPACK, D)

    cols = COLS // 4
    gfn = _make_gather(cols)
    y = _alloc_out()
    for h in range(4):
        g = gfn(xs[h * cols:(h + 1) * cols], tab_lin)
        g2 = g.reshape(-1).reshape(ROWS * cols * D // 128, 128)
        y = _finish_half(y, g2, h, cols)
    return jnp.transpose(y, (2, 0, 1))
